# baseline (speedup 1.0000x reference)
import dataclasses
import functools

import jax
import jax.numpy as jnp
from jax import lax
from jax.experimental import pallas as pl
from jax.experimental.pallas import tpu as pltpu
from jax.experimental.pallas import tpu_sc as plsc

_N = 10000
_E = 160000
_DIN = 256
_DE = 16
_H = 512
_L = 4

_NSUB = 16
_NCORE = 2
_EB = 64
_EPAD = 163840
_BLOCKS = _EPAD // (_NSUB * _EB)
_ACC_ROWS = 10048
_CHUNK = 128
_NCHUNK = _H // _CHUNK
_PK = _CHUNK // 2


def _pack_chunks(y):
    parts = []
    for cc in range(_NCHUNK):
        lo = y[:, cc * _CHUNK: cc * _CHUNK + _PK].astype(jnp.bfloat16)
        hi = y[:, cc * _CHUNK + _PK: (cc + 1) * _CHUNK].astype(jnp.bfloat16)
        lo32 = lax.bitcast_convert_type(lo, jnp.uint16).astype(jnp.int32)
        hi32 = lax.bitcast_convert_type(hi, jnp.uint16).astype(jnp.int32)
        parts.append(lo32 | (hi32 << 16))
    return jnp.stack(parts, axis=0)


def _edge_lin_body(x_ref, w_ref, b_ref, o_ref):
    y = jnp.dot(x_ref[...], w_ref[...], precision=lax.Precision.DEFAULT,
                preferred_element_type=jnp.float32) + b_ref[...]
    o_ref[...] = _pack_chunks(y)


def _edge_lin(x, w, b):
    R = 4096
    n, k = x.shape
    return pl.pallas_call(
        _edge_lin_body,
        grid=(n // R,),
        in_specs=[
            pl.BlockSpec((R, k), lambda i: (i, 0)),
            pl.BlockSpec((k, _H), lambda i: (0, 0)),
            pl.BlockSpec((1, _H), lambda i: (0, 0)),
        ],
        out_specs=pl.BlockSpec((_NCHUNK, R, _PK), lambda i: (0, i, 0)),
        out_shape=jax.ShapeDtypeStruct((_NCHUNK, n, _PK), jnp.int32),
    )(x, w, b.reshape(1, _H))


def _proj_body(x_ref, w_ref, b_ref, o_ref):
    o_ref[...] = jnp.dot(x_ref[...], w_ref[...], precision=lax.Precision.DEFAULT,
                         preferred_element_type=jnp.float32) + b_ref[...]


def _proj(x, w, b):
    R = 1000
    row = lambda i: (i, 0)
    return pl.pallas_call(
        _proj_body,
        grid=(_N // R,),
        in_specs=[
            pl.BlockSpec((R, _DIN), row),
            pl.BlockSpec((_DIN, _H), lambda i: (0, 0)),
            pl.BlockSpec((1, _H), lambda i: (0, 0)),
        ],
        out_specs=pl.BlockSpec((R, _H), row),
        out_shape=jax.ShapeDtypeStruct((_N, _H), jnp.float32),
    )(x, w, b.reshape(1, _H))


def _mlp_body(h_ref, a_ref, w1_ref, b1_ref, w2_ref, b2_ref, g_ref, bt_ref,
              hm_ref, ho_ref, hmo_ref):
    h = h_ref[...]
    m = h + a_ref[...]
    t = jnp.maximum(
        jnp.dot(m, w1_ref[...], precision=lax.Precision.DEFAULT,
                preferred_element_type=jnp.float32)
        + b1_ref[...], 0.0)
    hn = jnp.dot(t, w2_ref[...], precision=lax.Precision.DEFAULT,
              preferred_element_type=jnp.float32) + b2_ref[...]
    u = h + hn
    mu = jnp.mean(u, axis=-1, keepdims=True)
    var = jnp.mean((u - mu) ** 2, axis=-1, keepdims=True)
    y = (u - mu) * lax.rsqrt(var + 1e-5) * g_ref[...] + bt_ref[...]
    y = jnp.maximum(y, 0.0)
    ho_ref[...] = y
    hmo_ref[...] = jnp.maximum(hm_ref[...], y)


def _mlp(h, agg, w1, b1, w2, b2, g, bt, hmax):
    R = 1000
    row = lambda i: (i, 0)
    full = lambda i: (0, 0)
    return pl.pallas_call(
        _mlp_body,
        grid=(_N // R,),
        in_specs=[
            pl.BlockSpec((R, _H), row),
            pl.BlockSpec((R, _H), row),
            pl.BlockSpec((_H, _H), full),
            pl.BlockSpec((1, _H), full),
            pl.BlockSpec((_H, _H), full),
            pl.BlockSpec((1, _H), full),
            pl.BlockSpec((1, _H), full),
            pl.BlockSpec((1, _H), full),
            pl.BlockSpec((R, _H), row),
        ],
        out_specs=[pl.BlockSpec((R, _H), row), pl.BlockSpec((R, _H), row)],
        out_shape=[jax.ShapeDtypeStruct((_N, _H), jnp.float32),
                   jax.ShapeDtypeStruct((_N, _H), jnp.float32)],
    )(h, agg, w1, b1.reshape(1, _H), w2, b2.reshape(1, _H),
      g.reshape(1, _H), bt.reshape(1, _H), hmax)


def _unpack_lo(v):
    return plsc.bitcast(v << 16, jnp.float32)


def _unpack_hi(v):
    return plsc.bitcast(v & jnp.int32(-65536), jnp.float32)


def _sc_edge(h4, ep, sd):
    mesh = plsc.VectorSubcoreMesh(core_axis_name="c", subcore_axis_name="s")
    cp = pltpu.CompilerParams()
    if "needs_layout_passes" in pltpu.CompilerParams.__dataclass_fields__:
        cp = dataclasses.replace(cp, needs_layout_passes=False)

    @functools.partial(
        pl.kernel,
        out_type=jax.ShapeDtypeStruct((_N, _H), jnp.float32),
        mesh=mesh,
        compiler_params=cp,
        scratch_types=(
            [pltpu.VMEM((2, _EB), jnp.int32) for _ in range(2)]
            + [pltpu.VMEM((_EB,), jnp.int32) for _ in range(2)]
            + [pltpu.VMEM((_EB,), jnp.int32) for _ in range(4)]
            + [pltpu.VMEM((_EB, _PK), jnp.int32) for _ in range(2)]
            + [pltpu.VMEM((_EB, _CHUNK), jnp.float32) for _ in range(4)]
            + [pltpu.VMEM_SHARED((_ACC_ROWS, _CHUNK), jnp.float32)]
            + [pltpu.SemaphoreType.DMA for _ in range(8)]
        ),
    )
    def k(h4_hbm, ep_hbm, sd_hbm, agg_hbm,
          s0, s1, x0, x1, q0, q1, q2, q3, e0, e1,
          r0, r1, r2, r3, acc, si0, si1, sg0, sg1, ss0, ss1, ss2, ss3):
        sdv = [s0, s1]
        idxv = [x0, x1]
        scix = [q0, q1, q2, q3]
        ev = [e0, e1]
        rows = [r0, r1, r2, r3]
        sem_i = [si0, si1]
        sem_g = [sg0, sg1]
        sem_s = [ss0, ss1, ss2, ss3]

        cid = lax.axis_index("c")
        sid = lax.axis_index("s")
        epb = _EPAD // _NSUB
        wrows = 624

        def idx_start(bb, p):
            blk = sid * (epb // _EB) + bb
            pltpu.make_async_copy(sd_hbm.at[blk], sdv[p], sem_i[p]).start()

        def idx_wait(p):
            pltpu.make_async_copy(sd_hbm.at[0], sdv[p], sem_i[p]).wait()

        def ge_start(bb, p, r, c):
            base = pl.multiple_of(c * _EPAD + sid * epb + bb * _EB, _EB)
            pltpu.make_async_copy(
                ep_hbm.at[pl.ds(base, _EB)], ev[p], sem_g[p]).start()
            pltpu.make_async_copy(
                h4_hbm.at[idxv[p]], rows[r], sem_g[p]).start()

        def ge_wait(p, r):
            pltpu.make_async_copy(
                ep_hbm.at[pl.ds(0, _EB)], ev[p], sem_g[p]).wait()
            pltpu.make_async_copy(
                h4_hbm.at[idxv[p]], rows[r], sem_g[p]).wait()

        def sc_wait(r):
            pltpu.make_async_copy(rows[r], acc.at[scix[r]], sem_s[r]).wait()

        def compute_and_scatter(qq, rr):
            ge_wait(qq, rr)

            @plsc.parallel_loop(0, _EB, unroll=2)
            def _(row):
                for t in range(_PK // 16):
                    sl = pl.ds(t * 16, 16)
                    sh = pl.ds(_PK + t * 16, 16)
                    et = ev[qq].at[row][sl]
                    rows[rr].at[row][sl] = jnp.maximum(
                        rows[rr].at[row][sl] + _unpack_lo(et), 0.0)
                    rows[rr].at[row][sh] = jnp.maximum(
                        rows[rr].at[row][sh] + _unpack_hi(et), 0.0)

            pltpu.async_copy(rows[rr], acc.at[scix[rr]], sem_s[rr], add=True)

        for j in range(_NCHUNK // _NCORE):
            c = cid * (_NCHUNK // _NCORE) + j

            @pl.loop(0, _EB)
            def _(r):
                for g in range(_CHUNK // 16):
                    rows[0].at[r][pl.ds(g * 16, 16)] = jnp.zeros(
                        (16,), jnp.float32)

            ztiles = _ACC_ROWS // _EB
            for t in range(-(-ztiles // _NSUB)):
                tile = sid + t * _NSUB

                @pl.when(tile < ztiles)
                def _():
                    pltpu.make_async_copy(
                        rows[0], acc.at[pl.ds(tile * _EB, _EB)],
                        sem_s[0]).start()
            for t in range(-(-ztiles // _NSUB)):
                tile = sid + t * _NSUB

                @pl.when(tile < ztiles)
                def _():
                    pltpu.make_async_copy(
                        rows[0], acc.at[pl.ds(tile * _EB, _EB)],
                        sem_s[0]).wait()
            plsc.subcore_barrier()

            idx_start(0, 0)
            idx_start(1, 1)

            @pl.loop(0, _BLOCKS // 4)
            def _(grp):
                for u in range(4):
                    p = u % 2
                    q = 1 - p
                    rp = (u - 1) % 4
                    bb = grp * 4 + u
                    idx_wait(p)

                    @pl.when(grp >= 1)
                    def _():
                        sc_wait(u)

                    for t in range(_EB // 16):
                        s = pl.ds(t * 16, 16)
                        idxv[p][s] = sdv[p].at[0][s] * _NCHUNK + c
                        scix[u][s] = sdv[p].at[1][s]
                    ge_start(bb, p, u, c)
                    if u < 2:
                        idx_start(bb + 2, p)
                    else:
                        @pl.when(grp < _BLOCKS // 4 - 1)
                        def _():
                            idx_start(bb + 2, p)

                    if u == 0:
                        @pl.when(grp >= 1)
                        def _():
                            compute_and_scatter(q, rp)
                    else:
                        compute_and_scatter(q, rp)

            compute_and_scatter((_BLOCKS - 1) % 2, (_BLOCKS - 1) % 4)
            for r in range(4):
                sc_wait(r)

            plsc.subcore_barrier()
            pltpu.sync_copy(
                acc.at[pl.ds(sid * wrows, wrows)],
                agg_hbm.at[pl.ds(sid * wrows, wrows),
                           pl.ds(c * _CHUNK, _CHUNK)])

            @pl.when(sid == _NSUB - 1)
            def _():
                pltpu.sync_copy(
                    acc.at[pl.ds(_NSUB * wrows, _N - _NSUB * wrows)],
                    agg_hbm.at[pl.ds(_NSUB * wrows, _N - _NSUB * wrows),
                               pl.ds(c * _CHUNK, _CHUNK)])

            plsc.subcore_barrier()

    return k(h4, ep, sd)


def kernel(x, edge_index, edge_attr, Wp, bp, W1, b1, W2, b2, We, be, gamma, beta):
    src = edge_index[0]
    dst = edge_index[1]
    pad = _EPAD - _E
    srcp = jnp.concatenate([src, jnp.zeros((pad,), jnp.int32)])
    dstp = jnp.concatenate([dst, jnp.full((pad,), _N, jnp.int32)])
    sd = jnp.stack([srcp.reshape(-1, _EB), dstp.reshape(-1, _EB)], axis=1)
    eap = jnp.concatenate([edge_attr, jnp.zeros((pad, _DE), jnp.float32)], axis=0)

    h = _proj(x, Wp, bp)
    eps = [_edge_lin(eap, We[l], be[l]) for l in range(_L)]
    hmax = jnp.zeros((_N, _H), jnp.float32)
    for l in range(_L):
        agg = _sc_edge(h.reshape(_NCHUNK * _N, _CHUNK),
                       eps[l].reshape(_NCHUNK * _EPAD, _PK), sd)
        h, hmax = _mlp(h, agg, W1[l], b1[l], W2[l], b2[l],
                       gamma[l], beta[l], hmax)
    return hmax


# speedup vs baseline: 1.0187x; 1.0187x over previous
import dataclasses
import functools

import jax
import jax.numpy as jnp
from jax import lax
from jax.experimental import pallas as pl
from jax.experimental.pallas import tpu as pltpu
from jax.experimental.pallas import tpu_sc as plsc

_N = 10000
_E = 160000
_DIN = 256
_DE = 16
_H = 512
_L = 4

_NSUB = 16
_NCORE = 2
_EB = 64
_EPAD = 163840
_BLOCKS = _EPAD // (_NSUB * _EB)
_ACC_ROWS = 10048
_CHUNK = 128
_NCHUNK = _H // _CHUNK
_PK = _CHUNK // 2


def _pack_chunks(y):
    parts = []
    for cc in range(_NCHUNK):
        lo = y[:, cc * _CHUNK: cc * _CHUNK + _PK].astype(jnp.bfloat16)
        hi = y[:, cc * _CHUNK + _PK: (cc + 1) * _CHUNK].astype(jnp.bfloat16)
        lo32 = lax.bitcast_convert_type(lo, jnp.uint16).astype(jnp.int32)
        hi32 = lax.bitcast_convert_type(hi, jnp.uint16).astype(jnp.int32)
        parts.append(lo32 | (hi32 << 16))
    return jnp.stack(parts, axis=0)


def _edge_lin_body(x_ref, w_ref, b_ref, o_ref):
    y = jnp.dot(x_ref[...], w_ref[...],
                preferred_element_type=jnp.float32) + b_ref[...]
    o_ref[...] = _pack_chunks(y)


def _edge_lin(x, w, b):
    R = 4096
    n, k = x.shape
    return pl.pallas_call(
        _edge_lin_body,
        grid=(n // R,),
        in_specs=[
            pl.BlockSpec((R, k), lambda i: (i, 0)),
            pl.BlockSpec((k, _H), lambda i: (0, 0)),
            pl.BlockSpec((1, _H), lambda i: (0, 0)),
        ],
        out_specs=pl.BlockSpec((_NCHUNK, R, _PK), lambda i: (0, i, 0)),
        out_shape=jax.ShapeDtypeStruct((_NCHUNK, n, _PK), jnp.int32),
    )(x, w, b.reshape(1, _H))


def _proj_body(x_ref, w_ref, b_ref, o_ref):
    o_ref[...] = jnp.dot(x_ref[...], w_ref[...],
                         preferred_element_type=jnp.float32) + b_ref[...]


def _proj(x, w, b):
    R = 1000
    row = lambda i: (i, 0)
    return pl.pallas_call(
        _proj_body,
        grid=(_N // R,),
        in_specs=[
            pl.BlockSpec((R, _DIN), row),
            pl.BlockSpec((_DIN, _H), lambda i: (0, 0)),
            pl.BlockSpec((1, _H), lambda i: (0, 0)),
        ],
        out_specs=pl.BlockSpec((R, _H), row),
        out_shape=jax.ShapeDtypeStruct((_N, _H), jnp.float32),
    )(x, w, b.reshape(1, _H))


def _mlp_body(h_ref, a_ref, w1_ref, b1_ref, w2_ref, b2_ref, g_ref, bt_ref,
              hm_ref, ho_ref, hmo_ref):
    h = h_ref[...]
    m = h + a_ref[...]
    t = jnp.maximum(
        jnp.dot(m, w1_ref[...], preferred_element_type=jnp.float32)
        + b1_ref[...], 0.0)
    hn = jnp.dot(t, w2_ref[...], preferred_element_type=jnp.float32) + b2_ref[...]
    u = h + hn
    mu = jnp.mean(u, axis=-1, keepdims=True)
    var = jnp.mean((u - mu) ** 2, axis=-1, keepdims=True)
    y = (u - mu) * lax.rsqrt(var + 1e-5) * g_ref[...] + bt_ref[...]
    y = jnp.maximum(y, 0.0)
    ho_ref[...] = y
    hmo_ref[...] = jnp.maximum(hm_ref[...], y)


def _mlp(h, agg, w1, b1, w2, b2, g, bt, hmax):
    R = 1000
    row = lambda i: (i, 0)
    full = lambda i: (0, 0)
    return pl.pallas_call(
        _mlp_body,
        grid=(_N // R,),
        in_specs=[
            pl.BlockSpec((R, _H), row),
            pl.BlockSpec((R, _H), row),
            pl.BlockSpec((_H, _H), full),
            pl.BlockSpec((1, _H), full),
            pl.BlockSpec((_H, _H), full),
            pl.BlockSpec((1, _H), full),
            pl.BlockSpec((1, _H), full),
            pl.BlockSpec((1, _H), full),
            pl.BlockSpec((R, _H), row),
        ],
        out_specs=[pl.BlockSpec((R, _H), row), pl.BlockSpec((R, _H), row)],
        out_shape=[jax.ShapeDtypeStruct((_N, _H), jnp.float32),
                   jax.ShapeDtypeStruct((_N, _H), jnp.float32)],
    )(h, agg, w1, b1.reshape(1, _H), w2, b2.reshape(1, _H),
      g.reshape(1, _H), bt.reshape(1, _H), hmax)


def _unpack_lo(v):
    return plsc.bitcast(v << 16, jnp.float32)


def _unpack_hi(v):
    return plsc.bitcast(v & jnp.int32(-65536), jnp.float32)


def _sc_edge(h4, ep, srcp, dstp):
    mesh = plsc.VectorSubcoreMesh(core_axis_name="c", subcore_axis_name="s")
    cp = pltpu.CompilerParams()
    if "needs_layout_passes" in pltpu.CompilerParams.__dataclass_fields__:
        cp = dataclasses.replace(cp, needs_layout_passes=False)

    @functools.partial(
        pl.kernel,
        out_type=jax.ShapeDtypeStruct((_N, _H), jnp.float32),
        mesh=mesh,
        compiler_params=cp,
        scratch_types=(
            [pltpu.VMEM((_EB,), jnp.int32) for _ in range(2)]
            + [pltpu.VMEM((_EB,), jnp.int32) for _ in range(2)]
            + [pltpu.VMEM((_EB,), jnp.int32) for _ in range(2)]
            + [pltpu.VMEM((_EB,), jnp.int32) for _ in range(4)]
            + [pltpu.VMEM((_EB, _PK), jnp.int32) for _ in range(2)]
            + [pltpu.VMEM((_EB, _CHUNK), jnp.float32) for _ in range(4)]
            + [pltpu.VMEM_SHARED((_ACC_ROWS, _CHUNK), jnp.float32)]
            + [pltpu.SemaphoreType.DMA for _ in range(8)]
        ),
    )
    def k(h4_hbm, ep_hbm, src_hbm, dst_hbm, agg_hbm,
          s0, s1, d0, d1, x0, x1, q0, q1, q2, q3, e0, e1,
          r0, r1, r2, r3, acc, si0, si1, sg0, sg1, ss0, ss1, ss2, ss3):
        srcv = [s0, s1]
        dstv = [d0, d1]
        idxv = [x0, x1]
        scix = [q0, q1, q2, q3]
        ev = [e0, e1]
        rows = [r0, r1, r2, r3]
        sem_i = [si0, si1]
        sem_g = [sg0, sg1]
        sem_s = [ss0, ss1, ss2, ss3]

        cid = lax.axis_index("c")
        sid = lax.axis_index("s")
        epb = _EPAD // _NSUB
        wrows = 624

        def idx_start(bb, p):
            base = pl.multiple_of(sid * epb + bb * _EB, _EB)
            pltpu.make_async_copy(
                src_hbm.at[pl.ds(base, _EB)], srcv[p], sem_i[p]).start()
            pltpu.make_async_copy(
                dst_hbm.at[pl.ds(base, _EB)], dstv[p], sem_i[p]).start()

        def idx_wait(p):
            pltpu.make_async_copy(
                src_hbm.at[pl.ds(0, _EB)], srcv[p], sem_i[p]).wait()
            pltpu.make_async_copy(
                dst_hbm.at[pl.ds(0, _EB)], dstv[p], sem_i[p]).wait()

        def ge_start(bb, p, r, c):
            base = pl.multiple_of(c * _EPAD + sid * epb + bb * _EB, _EB)
            pltpu.make_async_copy(
                ep_hbm.at[pl.ds(base, _EB)], ev[p], sem_g[p]).start()
            pltpu.make_async_copy(
                h4_hbm.at[idxv[p]], rows[r], sem_g[p]).start()

        def ge_wait(p, r):
            pltpu.make_async_copy(
                ep_hbm.at[pl.ds(0, _EB)], ev[p], sem_g[p]).wait()
            pltpu.make_async_copy(
                h4_hbm.at[idxv[p]], rows[r], sem_g[p]).wait()

        def sc_wait(r):
            pltpu.make_async_copy(rows[r], acc.at[scix[r]], sem_s[r]).wait()

        def compute_and_scatter(qq, rr):
            ge_wait(qq, rr)

            @pl.loop(0, _EB)
            def _(row):
                for t in range(_PK // 16):
                    sl = pl.ds(t * 16, 16)
                    sh = pl.ds(_PK + t * 16, 16)
                    et = ev[qq].at[row][sl]
                    rows[rr].at[row][sl] = jnp.maximum(
                        rows[rr].at[row][sl] + _unpack_lo(et), 0.0)
                    rows[rr].at[row][sh] = jnp.maximum(
                        rows[rr].at[row][sh] + _unpack_hi(et), 0.0)

            pltpu.async_copy(rows[rr], acc.at[scix[rr]], sem_s[rr], add=True)

        for j in range(_NCHUNK // _NCORE):
            c = cid * (_NCHUNK // _NCORE) + j

            @pl.loop(0, _EB)
            def _(r):
                for g in range(_CHUNK // 16):
                    rows[0].at[r][pl.ds(g * 16, 16)] = jnp.zeros(
                        (16,), jnp.float32)

            ztiles = _ACC_ROWS // _EB
            for t in range(-(-ztiles // _NSUB)):
                tile = sid + t * _NSUB

                @pl.when(tile < ztiles)
                def _():
                    pltpu.sync_copy(rows[0], acc.at[pl.ds(tile * _EB, _EB)])
            plsc.subcore_barrier()

            idx_start(0, 0)
            idx_start(1, 1)

            @pl.loop(0, _BLOCKS // 4)
            def _(grp):
                for u in range(4):
                    p = u % 2
                    q = 1 - p
                    rp = (u - 1) % 4
                    bb = grp * 4 + u
                    idx_wait(p)

                    @pl.when(grp >= 1)
                    def _():
                        sc_wait(u)

                    for t in range(_EB // 16):
                        s = pl.ds(t * 16, 16)
                        idxv[p][s] = srcv[p][s] * _NCHUNK + c
                        scix[u][s] = dstv[p][s]
                    ge_start(bb, p, u, c)
                    if u < 2:
                        idx_start(bb + 2, p)
                    else:
                        @pl.when(grp < _BLOCKS // 4 - 1)
                        def _():
                            idx_start(bb + 2, p)

                    if u == 0:
                        @pl.when(grp >= 1)
                        def _():
                            compute_and_scatter(q, rp)
                    else:
                        compute_and_scatter(q, rp)

            compute_and_scatter((_BLOCKS - 1) % 2, (_BLOCKS - 1) % 4)
            for r in range(4):
                sc_wait(r)

            plsc.subcore_barrier()
            pltpu.sync_copy(
                acc.at[pl.ds(sid * wrows, wrows)],
                agg_hbm.at[pl.ds(sid * wrows, wrows),
                           pl.ds(c * _CHUNK, _CHUNK)])

            @pl.when(sid == _NSUB - 1)
            def _():
                pltpu.sync_copy(
                    acc.at[pl.ds(_NSUB * wrows, _N - _NSUB * wrows)],
                    agg_hbm.at[pl.ds(_NSUB * wrows, _N - _NSUB * wrows),
                               pl.ds(c * _CHUNK, _CHUNK)])

            plsc.subcore_barrier()

    return k(h4, ep, srcp, dstp)


def kernel(x, edge_index, edge_attr, Wp, bp, W1, b1, W2, b2, We, be, gamma, beta):
    src = edge_index[0]
    dst = edge_index[1]
    pad = _EPAD - _E
    srcp = jnp.concatenate([src, jnp.zeros((pad,), jnp.int32)])
    dstp = jnp.concatenate([dst, jnp.full((pad,), _N, jnp.int32)])
    eap = jnp.concatenate([edge_attr, jnp.zeros((pad, _DE), jnp.float32)], axis=0)

    h = _proj(x, Wp, bp)
    eps = [_edge_lin(eap, We[l], be[l]) for l in range(_L)]
    hmax = jnp.zeros((_N, _H), jnp.float32)
    for l in range(_L):
        agg = _sc_edge(h.reshape(_NCHUNK * _N, _CHUNK),
                       eps[l].reshape(_NCHUNK * _EPAD, _PK), srcp, dstp)
        h, hmax = _mlp(h, agg, W1[l], b1[l], W2[l], b2[l],
                       gamma[l], beta[l], hmax)
    return hmax
